# baseline (speedup 1.0000x reference)
import functools

import jax
import jax.numpy as jnp
from jax import lax
from jax.experimental import pallas as pl
from jax.experimental.pallas import tpu as pltpu
from jax.experimental.pallas import tpu_sc as plsc

_BATCH = 4096
_HIST = 50
_EMB = 32
_NW = 32
_BT = _BATCH // _NW
_L = 16


def _transpose_block(rows_ref, tout_ref, iota):
    for f in range(_EMB):
        col = jnp.full((_L,), f, jnp.int32)
        for gidx in range(_BT // _L):
            rid = iota + (gidx * _L)
            v = plsc.load_gather(rows_ref, [rid, col])
            tout_ref[f // 8, f % 8, pl.ds(gidx * _L, _L)] = v


def _make_sc_gather():
    mesh = plsc.VectorSubcoreMesh(core_axis_name="c", subcore_axis_name="s")

    @functools.partial(
        pl.kernel,
        mesh=mesh,
        out_type=jax.ShapeDtypeStruct((_HIST, 4, _NW, 8, _BT), jnp.float32),
        scratch_types=[
            pltpu.VMEM((_HIST, _BT), jnp.int32),
            pltpu.VMEM((2, _BT, _EMB), jnp.float32),
            pltpu.VMEM((2, 4, 8, _BT), jnp.float32),
            pltpu.SemaphoreType.DMA,
            pltpu.SemaphoreType.DMA,
            pltpu.SemaphoreType.DMA,
            pltpu.SemaphoreType.DMA,
        ],
        compiler_params=pltpu.CompilerParams(
            use_tc_tiling_on_sc=False, needs_layout_passes=False
        ),
    )
    def sc_gather(idx_hbm, tab_hbm, out_hbm, idx_v, rows_v, tout_v,
                  g0, g1, w0, w1):
        w = lax.axis_index("s") * 2 + lax.axis_index("c")
        iota = lax.iota(jnp.int32, _L)
        pltpu.sync_copy(idx_hbm.at[:, pl.ds(w * _BT, _BT)], idx_v)

        def gather(h, buf, sem):
            pltpu.async_copy(tab_hbm.at[idx_v.at[h]], rows_v.at[buf], sem)

        def wait_gather(buf, sem):
            pltpu.make_async_copy(
                tab_hbm.at[idx_v.at[0]], rows_v.at[buf], sem
            ).wait()

        def write(h, buf, sem):
            pltpu.async_copy(tout_v.at[buf], out_hbm.at[h].at[:, w], sem)

        def wait_write(buf, sem):
            pltpu.make_async_copy(
                tout_v.at[buf], out_hbm.at[0].at[:, w], sem
            ).wait()

        def half(g, h, buf, gsem, wsem):
            wait_gather(buf, gsem)

            @pl.when(g >= 1)
            def _():
                wait_write(buf, wsem)

            _transpose_block(rows_v.at[buf], tout_v.at[buf], iota)

            @pl.when(g < _HIST // 2 - 1)
            def _():
                gather(h + 2, buf, gsem)

            write(h, buf, wsem)

        def group(g, carry):
            h0 = 2 * g
            half(g, h0, 0, g0, w0)
            half(g, h0 + 1, 1, g1, w1)
            return carry

        gather(0, 0, g0)
        gather(1, 1, g1)
        lax.fori_loop(0, _HIST // 2, group, 0)
        wait_write(0, w0)
        wait_write(1, w1)

    return sc_gather


def kernel(x, table):
    idx = x.T.reshape(_HIST, _BATCH).astype(jnp.int32)
    out5 = _make_sc_gather()(idx, table)
    return out5.transpose(2, 4, 0, 1, 3).reshape(_BATCH, _HIST, _EMB)


# speedup vs baseline: 1.0564x; 1.0564x over previous
import functools

import jax
import jax.numpy as jnp
from jax import lax
from jax.experimental import pallas as pl
from jax.experimental.pallas import tpu as pltpu
from jax.experimental.pallas import tpu_sc as plsc

_BATCH = 4096
_HIST = 50
_EMB = 32
_NW = 32
_BT = _BATCH // _NW
_NBUF = 3


def _make_sc_gather():
    mesh = plsc.VectorSubcoreMesh(core_axis_name="c", subcore_axis_name="s")

    @functools.partial(
        pl.kernel,
        mesh=mesh,
        out_type=jax.ShapeDtypeStruct((_BATCH, _HIST, _EMB), jnp.float32),
        scratch_types=[
            pltpu.VMEM((_HIST, _BT), jnp.int32),
            pltpu.VMEM((_NBUF, _BT, _EMB), jnp.float32),
        ]
        + [pltpu.SemaphoreType.DMA] * (2 * _NBUF),
        compiler_params=pltpu.CompilerParams(
            use_tc_tiling_on_sc=False, needs_layout_passes=False
        ),
    )
    def sc_gather(idx_hbm, tab_hbm, out_hbm, idx_v, rows_v, *sems):
        gsem = sems[:_NBUF]
        wsem = sems[_NBUF:]
        w = lax.axis_index("s") * 2 + lax.axis_index("c")
        pltpu.sync_copy(idx_hbm.at[:, pl.ds(w * _BT, _BT)], idx_v)

        def gather(h):
            pltpu.async_copy(
                tab_hbm.at[idx_v.at[h]], rows_v.at[h % _NBUF], gsem[h % _NBUF]
            )

        def wait_gather(h):
            pltpu.make_async_copy(
                tab_hbm.at[idx_v.at[h]], rows_v.at[h % _NBUF], gsem[h % _NBUF]
            ).wait()

        def write(h):
            pltpu.async_copy(
                rows_v.at[h % _NBUF],
                out_hbm.at[pl.ds(w * _BT, _BT), h],
                wsem[h % _NBUF],
            )

        def wait_write(h):
            pltpu.make_async_copy(
                rows_v.at[h % _NBUF],
                out_hbm.at[pl.ds(w * _BT, _BT), h],
                wsem[h % _NBUF],
            ).wait()

        gather(0)
        gather(1)
        for h in range(_HIST):
            if h + 2 < _HIST:
                if h >= 1:
                    wait_write(h - 1)
                gather(h + 2)
            wait_gather(h)
            write(h)
        wait_write(_HIST - 2)
        wait_write(_HIST - 1)

    return sc_gather


def kernel(x, table):
    idx = x.T.reshape(_HIST, _BATCH).astype(jnp.int32)
    return _make_sc_gather()(idx, table)
